# Initial kernel scaffold; baseline (speedup 1.0000x reference)
import functools

import jax
import jax.numpy as jnp
from jax import lax
from jax.experimental import pallas as pl
from jax.experimental.pallas import tpu as pltpu
from jax.experimental.pallas import tpu_sc as plsc

N = 10000
D = 128
G = 512
E = 320000
NP = 10240
ACC = 10496
EPAD = 327680
EPT = EPAD // 32
CH = 128
NC, NS = 2, 16
BLK = 256
NBLK = NP // BLK


def _sc_mesh():
    return plsc.VectorSubcoreMesh(core_axis_name="c", subcore_axis_name="s")


def _stats_body(dst_hbm, batch_hbm, dinv_hbm, cnt_hbm,
                deg_sh, cnt_sh, idx_v, ones_v, work_v, cbuf):
    c = lax.axis_index("c")
    s = lax.axis_index("s")

    def fill_ones(i, _):
        ones_v[pl.ds(i * 16, 16)] = jnp.full((16,), 1.0, jnp.float32)
        return 0
    lax.fori_loop(0, 656 // 16, fill_ones, 0)

    def fill_zero(i, _):
        work_v[pl.ds(i * 16, 16)] = jnp.zeros((16,), jnp.float32)
        return 0
    lax.fori_loop(0, 320 // 16, fill_zero, 0)

    pltpu.sync_copy(ones_v, deg_sh.at[pl.ds(s * 656, 656)])
    pltpu.sync_copy(work_v.at[pl.ds(0, 64)], cnt_sh.at[pl.ds(s * 64, 64)])
    plsc.subcore_barrier()

    def deg_body(j, _):
        b = s * (EPAD // NS) + j * CH
        pltpu.sync_copy(dst_hbm.at[pl.ds(b, CH)], idx_v)
        pltpu.sync_copy(ones_v.at[pl.ds(0, CH)], deg_sh.at[idx_v], add=True)
        return 0
    lax.fori_loop(0, EPAD // NS // CH, deg_body, 0)

    def cnt_body(j, _):
        b = s * (NP // NS) + j * CH
        pltpu.sync_copy(batch_hbm.at[pl.ds(b, CH)], idx_v)
        pltpu.sync_copy(ones_v.at[pl.ds(0, CH)], cnt_sh.at[idx_v], add=True)
        return 0
    lax.fori_loop(0, NP // NS // CH, cnt_body, 0)
    plsc.subcore_barrier()

    off = c * (NP // NC) + s * (NP // NC // NS)
    pltpu.sync_copy(deg_sh.at[pl.ds(off, 320)], work_v)

    def rs_body(k, _):
        v = work_v[pl.ds(k * 16, 16)]
        i = plsc.bitcast(v, jnp.int32)
        i = jnp.int32(0x5F3759DF) - lax.shift_right_logical(i, 1)
        y = plsc.bitcast(i, jnp.float32)
        h = v * jnp.float32(0.5)
        y = y * (jnp.float32(1.5) - h * y * y)
        y = y * (jnp.float32(1.5) - h * y * y)
        y = y * (jnp.float32(1.5) - h * y * y)
        work_v[pl.ds(k * 16, 16)] = y
        return 0
    lax.fori_loop(0, 320 // 16, rs_body, 0)
    pltpu.sync_copy(work_v, dinv_hbm.at[pl.ds(off, 320)])

    off2 = c * (G // NC) + s * (G // NC // NS)
    pltpu.sync_copy(cnt_sh.at[pl.ds(off2, 16)], cbuf)
    pltpu.sync_copy(cbuf, cnt_hbm.at[pl.ds(off2, 16)])


def _stats(dst_p, batch_p):
    k = pl.kernel(
        _stats_body,
        out_type=(jax.ShapeDtypeStruct((NP,), jnp.float32),
                  jax.ShapeDtypeStruct((G,), jnp.float32)),
        mesh=_sc_mesh(),
        scratch_types=[
            pltpu.VMEM_SHARED((ACC,), jnp.float32),
            pltpu.VMEM_SHARED((1024,), jnp.float32),
            pltpu.VMEM((CH,), jnp.int32),
            pltpu.VMEM((656,), jnp.float32),
            pltpu.VMEM((320,), jnp.float32),
            pltpu.VMEM((16,), jnp.float32),
        ],
    )
    return k(dst_p, batch_p)


def _scatter_body(h_hbm, src_hbm, dst_hbm, out_hbm,
                  acc_sh, srcv, dstv, rows_v, sem):
    c = lax.axis_index("c")
    s = lax.axis_index("s")

    def zr(r, _):
        for k in range(8):
            rows_v[r, pl.ds(k * 16, 16)] = jnp.zeros((16,), jnp.float32)
        return 0
    lax.fori_loop(0, CH, zr, 0)

    base = s * (ACC // NS)
    for k in range(5):
        pltpu.sync_copy(rows_v, acc_sh.at[pl.ds(base + k * 128, 128)])
    pltpu.sync_copy(rows_v.at[pl.ds(0, 16)], acc_sh.at[pl.ds(base + 640, 16)])
    plsc.subcore_barrier()

    estart = (c * NS + s) * EPT

    def body(j, _):
        b = estart + j * CH
        pltpu.sync_copy(src_hbm.at[pl.ds(b, CH)], srcv)
        pltpu.sync_copy(dst_hbm.at[pl.ds(b, CH)], dstv)
        pltpu.async_copy(h_hbm.at[srcv], rows_v, sem).wait()
        pltpu.sync_copy(rows_v, acc_sh.at[dstv], add=True)
        return 0
    lax.fori_loop(0, EPT // CH, body, 0)
    plsc.subcore_barrier()

    ab = s * (NP // NS)
    ob = c * NP + ab
    for k in range(5):
        pltpu.sync_copy(acc_sh.at[pl.ds(ab + k * 128, 128)], rows_v)
        pltpu.sync_copy(rows_v, out_hbm.at[pl.ds(ob + k * 128, 128)])


def _edge_scatter(h, src_p, dst_p):
    k = pl.kernel(
        _scatter_body,
        out_type=jax.ShapeDtypeStruct((NC * NP, 128), jnp.float32),
        mesh=_sc_mesh(),
        scratch_types=[
            pltpu.VMEM_SHARED((ACC, 128), jnp.float32),
            pltpu.VMEM((CH,), jnp.int32),
            pltpu.VMEM((CH,), jnp.int32),
            pltpu.VMEM((CH, 128), jnp.float32),
            pltpu.SemaphoreType.DMA,
        ],
    )
    return k(h, src_p, dst_p)


def _mm_scale_body(x_ref, w_ref, d_ref, o_ref):
    o_ref[...] = jnp.dot(x_ref[...], w_ref[...],
                         preferred_element_type=jnp.float32) * d_ref[...]


def _mm_scale(x_p, W, dinv2):
    return pl.pallas_call(
        _mm_scale_body,
        out_shape=jax.ShapeDtypeStruct((NP, 128), jnp.float32),
        grid=(NBLK,),
        in_specs=[pl.BlockSpec((BLK, 128), lambda i: (i, 0)),
                  pl.BlockSpec((128, 128), lambda i: (0, 0)),
                  pl.BlockSpec((BLK, 1), lambda i: (i, 0))],
        out_specs=pl.BlockSpec((BLK, 128), lambda i: (i, 0)),
    )(x_p, W, dinv2)


def _layer_mid_body(s0_ref, s1_ref, h_ref, d_ref, b_ref, w_ref, o_ref):
    i = pl.program_id(0)
    tot = s0_ref[...] + s1_ref[...] + h_ref[...]
    act = jnp.maximum(d_ref[...] * tot + b_ref[...], 0.0)
    row = i * BLK + lax.broadcasted_iota(jnp.int32, (BLK, 128), 0)
    act = jnp.where(row < N, act, 0.0)
    o_ref[...] = jnp.dot(act, w_ref[...],
                         preferred_element_type=jnp.float32) * d_ref[...]


def _layer_mid(s0, s1, ht, dinv2, b1r, W2):
    return pl.pallas_call(
        _layer_mid_body,
        out_shape=jax.ShapeDtypeStruct((NP, 128), jnp.float32),
        grid=(NBLK,),
        in_specs=[pl.BlockSpec((BLK, 128), lambda i: (i, 0)),
                  pl.BlockSpec((BLK, 128), lambda i: (i, 0)),
                  pl.BlockSpec((BLK, 128), lambda i: (i, 0)),
                  pl.BlockSpec((BLK, 1), lambda i: (i, 0)),
                  pl.BlockSpec((1, 128), lambda i: (0, 0)),
                  pl.BlockSpec((128, 128), lambda i: (0, 0))],
        out_specs=pl.BlockSpec((BLK, 128), lambda i: (i, 0)),
    )(s0, s1, ht, dinv2, b1r, W2)


def _finale_body(s0_ref, s1_ref, h_ref, d_ref, b_ref, bt_ref, cnt_ref,
                 wl_ref, bl_ref, wd1_ref, bd1_ref, wd2_ref, bd2_ref, o_ref):
    i = pl.program_id(0)
    tot = s0_ref[...] + s1_ref[...] + h_ref[...]
    act = jnp.maximum(d_ref[...] * tot + b_ref[...], 0.0)
    gid = lax.broadcasted_iota(jnp.float32, (BLK, G), 1)
    onehot = jnp.where(bt_ref[...] == gid, 1.0, 0.0)
    part = lax.dot_general(onehot, act, (((0,), (0,)), ((), ())),
                           preferred_element_type=jnp.float32)

    @pl.when(i == 0)
    def _():
        o_ref[...] = jnp.zeros_like(o_ref)

    o_ref[...] += part

    @pl.when(i == NBLK - 1)
    def _():
        pooled = o_ref[...] / jnp.maximum(cnt_ref[...], 1.0)
        lab = jax.nn.sigmoid(
            jnp.dot(pooled, wl_ref[...], preferred_element_type=jnp.float32)
            + bl_ref[...])
        dmid = jnp.maximum(
            jnp.dot(pooled, wd1_ref[...], preferred_element_type=jnp.float32)
            + bd1_ref[...], 0.0)
        dom = (jnp.dot(dmid, wd2_ref[...], preferred_element_type=jnp.float32)
               + bd2_ref[...])
        col = lax.broadcasted_iota(jnp.int32, (G, 128), 1)
        o_ref[...] = jnp.where(col == 0, lab, dom)


def _finale(s0, s1, ht, dinv2, b2r, batchf, cntr,
            Wlp, blp, Wd1, bd1r, Wd2p, bd2p):
    return pl.pallas_call(
        _finale_body,
        out_shape=jax.ShapeDtypeStruct((G, 128), jnp.float32),
        grid=(NBLK,),
        in_specs=[pl.BlockSpec((BLK, 128), lambda i: (i, 0)),
                  pl.BlockSpec((BLK, 128), lambda i: (i, 0)),
                  pl.BlockSpec((BLK, 128), lambda i: (i, 0)),
                  pl.BlockSpec((BLK, 1), lambda i: (i, 0)),
                  pl.BlockSpec((1, 128), lambda i: (0, 0)),
                  pl.BlockSpec((BLK, 1), lambda i: (i, 0)),
                  pl.BlockSpec((G, 1), lambda i: (0, 0)),
                  pl.BlockSpec((128, 128), lambda i: (0, 0)),
                  pl.BlockSpec((1, 128), lambda i: (0, 0)),
                  pl.BlockSpec((128, 64), lambda i: (0, 0)),
                  pl.BlockSpec((1, 64), lambda i: (0, 0)),
                  pl.BlockSpec((64, 128), lambda i: (0, 0)),
                  pl.BlockSpec((1, 128), lambda i: (0, 0))],
        out_specs=pl.BlockSpec((G, 128), lambda i: (0, 0)),
    )(s0, s1, ht, dinv2, b2r, batchf, cntr, Wlp, blp, Wd1, bd1r, Wd2p, bd2p)


def kernel(x, edge_index, batch, W1, b1, W2, b2, Wl, bl, Wd1, bd1, Wd2, bd2):
    npad = NP - N
    epad = EPAD - E
    src = edge_index[0]
    dst = edge_index[1]
    ar = jnp.arange(epad, dtype=jnp.int32)
    src_p = jnp.concatenate([src, N + ar % npad])
    dst_p = jnp.concatenate([dst, NP + ar % 16])
    batch_p = jnp.concatenate([batch, jnp.full((npad,), G, jnp.int32)])
    x_p = jnp.pad(x, ((0, npad), (0, 0)))

    dinv, cnt = _stats(dst_p, batch_p)
    dinv2 = dinv.reshape(NP, 1)

    h1t = _mm_scale(x_p, W1, dinv2)
    s1 = _edge_scatter(h1t, src_p, dst_p)
    h2t = _layer_mid(s1[:NP], s1[NP:], h1t, dinv2, b1[None, :], W2)
    s2 = _edge_scatter(h2t, src_p, dst_p)

    heads = _finale(
        s2[:NP], s2[NP:], h2t, dinv2, b2[None, :],
        batch_p.astype(jnp.float32).reshape(NP, 1), cnt.reshape(G, 1),
        jnp.pad(Wl, ((0, 0), (0, 127))), jnp.pad(bl[None, :], ((0, 0), (0, 127))),
        Wd1, bd1[None, :],
        jnp.pad(Wd2, ((0, 0), (1, 125))), jnp.pad(bd2[None, :], ((0, 0), (1, 125))),
    )
    return heads[:, 0:1], heads[:, 1:3]


# speedup vs baseline: 13.3818x; 13.3818x over previous
import functools

import jax
import jax.numpy as jnp
from jax import lax
from jax.experimental import pallas as pl
from jax.experimental.pallas import tpu as pltpu
from jax.experimental.pallas import tpu_sc as plsc

N = 10000
D = 128
G = 512
E = 320000
NP = 10240
ACC = 10496
EPAD = 327680
EPT = EPAD // 32
CH = 128
NC, NS = 2, 16
BLK = 256
NBLK = NP // BLK


def _sc_mesh():
    return plsc.VectorSubcoreMesh(core_axis_name="c", subcore_axis_name="s")


def _stats_body(dst_hbm, batch_hbm, dinv_hbm, cnt_hbm,
                deg_sh, cnt_sh, idx_v, ones_v, work_v, cbuf):
    c = lax.axis_index("c")
    s = lax.axis_index("s")

    def fill_ones(i, _):
        ones_v[pl.ds(i * 16, 16)] = jnp.full((16,), 1.0, jnp.float32)
        return 0
    lax.fori_loop(0, 656 // 16, fill_ones, 0)

    def fill_zero(i, _):
        work_v[pl.ds(i * 16, 16)] = jnp.zeros((16,), jnp.float32)
        return 0
    lax.fori_loop(0, 320 // 16, fill_zero, 0)

    pltpu.sync_copy(ones_v, deg_sh.at[pl.ds(s * 656, 656)])
    pltpu.sync_copy(work_v.at[pl.ds(0, 64)], cnt_sh.at[pl.ds(s * 64, 64)])
    plsc.subcore_barrier()

    def deg_body(j, _):
        b = s * (EPAD // NS) + j * CH
        pltpu.sync_copy(dst_hbm.at[pl.ds(b, CH)], idx_v)
        pltpu.sync_copy(ones_v.at[pl.ds(0, CH)], deg_sh.at[idx_v], add=True)
        return 0
    lax.fori_loop(0, EPAD // NS // CH, deg_body, 0)

    def cnt_body(j, _):
        b = s * (NP // NS) + j * CH
        pltpu.sync_copy(batch_hbm.at[pl.ds(b, CH)], idx_v)
        pltpu.sync_copy(ones_v.at[pl.ds(0, CH)], cnt_sh.at[idx_v], add=True)
        return 0
    lax.fori_loop(0, NP // NS // CH, cnt_body, 0)
    plsc.subcore_barrier()

    off = c * (NP // NC) + s * (NP // NC // NS)
    pltpu.sync_copy(deg_sh.at[pl.ds(off, 320)], work_v)
    pltpu.sync_copy(work_v, dinv_hbm.at[pl.ds(off, 320)])

    off2 = c * (G // NC) + s * (G // NC // NS)
    pltpu.sync_copy(cnt_sh.at[pl.ds(off2, 16)], cbuf)
    pltpu.sync_copy(cbuf, cnt_hbm.at[pl.ds(off2, 16)])


def _stats(dst_p, batch_p):
    k = pl.kernel(
        _stats_body,
        out_type=(jax.ShapeDtypeStruct((NP,), jnp.float32),
                  jax.ShapeDtypeStruct((G,), jnp.float32)),
        mesh=_sc_mesh(),
        scratch_types=[
            pltpu.VMEM_SHARED((ACC,), jnp.float32),
            pltpu.VMEM_SHARED((1024,), jnp.float32),
            pltpu.VMEM((CH,), jnp.int32),
            pltpu.VMEM((656,), jnp.float32),
            pltpu.VMEM((320,), jnp.float32),
            pltpu.VMEM((16,), jnp.float32),
        ],
    )
    return k(dst_p, batch_p)


def _scatter_body(h_hbm, src_hbm, dst_hbm, out_hbm,
                  acc_sh, srcv, dstv, rows_v, sem):
    c = lax.axis_index("c")
    s = lax.axis_index("s")

    def zr(r, _):
        for k in range(8):
            rows_v[r, pl.ds(k * 16, 16)] = jnp.zeros((16,), jnp.float32)
        return 0
    lax.fori_loop(0, CH, zr, 0)

    base = s * (ACC // NS)
    for k in range(5):
        pltpu.sync_copy(rows_v, acc_sh.at[pl.ds(base + k * 128, 128)])
    pltpu.sync_copy(rows_v.at[pl.ds(0, 16)], acc_sh.at[pl.ds(base + 640, 16)])
    plsc.subcore_barrier()

    estart = (c * NS + s) * EPT

    def body(j, _):
        b = estart + j * CH
        pltpu.sync_copy(src_hbm.at[pl.ds(b, CH)], srcv)
        pltpu.sync_copy(dst_hbm.at[pl.ds(b, CH)], dstv)
        pltpu.async_copy(h_hbm.at[srcv], rows_v, sem).wait()
        pltpu.sync_copy(rows_v, acc_sh.at[dstv], add=True)
        return 0
    lax.fori_loop(0, EPT // CH, body, 0)
    plsc.subcore_barrier()

    ab = s * (NP // NS)
    ob = c * NP + ab
    for k in range(5):
        pltpu.sync_copy(acc_sh.at[pl.ds(ab + k * 128, 128)], rows_v)
        pltpu.sync_copy(rows_v, out_hbm.at[pl.ds(ob + k * 128, 128)])


def _edge_scatter(h, src_p, dst_p):
    k = pl.kernel(
        _scatter_body,
        out_type=jax.ShapeDtypeStruct((NC * NP, 128), jnp.float32),
        mesh=_sc_mesh(),
        scratch_types=[
            pltpu.VMEM_SHARED((ACC, 128), jnp.float32),
            pltpu.VMEM((CH,), jnp.int32),
            pltpu.VMEM((CH,), jnp.int32),
            pltpu.VMEM((CH, 128), jnp.float32),
            pltpu.SemaphoreType.DMA,
        ],
    )
    return k(h, src_p, dst_p)


def _rsqrt_body(d_ref, o_ref):
    o_ref[...] = lax.rsqrt(d_ref[...])


def _rsqrt_tc(deg2):
    return pl.pallas_call(
        _rsqrt_body,
        out_shape=jax.ShapeDtypeStruct((NP, 1), jnp.float32),
        grid=(NBLK,),
        in_specs=[pl.BlockSpec((BLK, 1), lambda i: (i, 0))],
        out_specs=pl.BlockSpec((BLK, 1), lambda i: (i, 0)),
    )(deg2)


def _mm_scale_body(x_ref, w_ref, d_ref, o_ref):
    o_ref[...] = jnp.dot(x_ref[...], w_ref[...],
                         preferred_element_type=jnp.float32) * d_ref[...]


def _mm_scale(x_p, W, dinv2):
    return pl.pallas_call(
        _mm_scale_body,
        out_shape=jax.ShapeDtypeStruct((NP, 128), jnp.float32),
        grid=(NBLK,),
        in_specs=[pl.BlockSpec((BLK, 128), lambda i: (i, 0)),
                  pl.BlockSpec((128, 128), lambda i: (0, 0)),
                  pl.BlockSpec((BLK, 1), lambda i: (i, 0))],
        out_specs=pl.BlockSpec((BLK, 128), lambda i: (i, 0)),
    )(x_p, W, dinv2)


def _layer_mid_body(s0_ref, s1_ref, h_ref, d_ref, b_ref, w_ref, o_ref):
    i = pl.program_id(0)
    tot = s0_ref[...] + s1_ref[...] + h_ref[...]
    act = jnp.maximum(d_ref[...] * tot + b_ref[...], 0.0)
    row = i * BLK + lax.broadcasted_iota(jnp.int32, (BLK, 128), 0)
    act = jnp.where(row < N, act, 0.0)
    o_ref[...] = jnp.dot(act, w_ref[...],
                         preferred_element_type=jnp.float32) * d_ref[...]


def _layer_mid(s0, s1, ht, dinv2, b1r, W2):
    return pl.pallas_call(
        _layer_mid_body,
        out_shape=jax.ShapeDtypeStruct((NP, 128), jnp.float32),
        grid=(NBLK,),
        in_specs=[pl.BlockSpec((BLK, 128), lambda i: (i, 0)),
                  pl.BlockSpec((BLK, 128), lambda i: (i, 0)),
                  pl.BlockSpec((BLK, 128), lambda i: (i, 0)),
                  pl.BlockSpec((BLK, 1), lambda i: (i, 0)),
                  pl.BlockSpec((1, 128), lambda i: (0, 0)),
                  pl.BlockSpec((128, 128), lambda i: (0, 0))],
        out_specs=pl.BlockSpec((BLK, 128), lambda i: (i, 0)),
    )(s0, s1, ht, dinv2, b1r, W2)


def _finale_body(s0_ref, s1_ref, h_ref, d_ref, b_ref, bt_ref, cnt_ref,
                 wl_ref, bl_ref, wd1_ref, bd1_ref, wd2_ref, bd2_ref, o_ref):
    i = pl.program_id(0)
    tot = s0_ref[...] + s1_ref[...] + h_ref[...]
    act = jnp.maximum(d_ref[...] * tot + b_ref[...], 0.0)
    gid = lax.broadcasted_iota(jnp.int32, (BLK, G), 1)
    onehot = jnp.where(bt_ref[...].astype(jnp.int32) == gid, 1.0, 0.0)
    part = lax.dot_general(onehot, act, (((0,), (0,)), ((), ())),
                           preferred_element_type=jnp.float32)

    @pl.when(i == 0)
    def _():
        o_ref[...] = jnp.zeros_like(o_ref)

    o_ref[...] += part

    @pl.when(i == NBLK - 1)
    def _():
        pooled = o_ref[...] / jnp.maximum(cnt_ref[...], 1.0)
        lab = jax.nn.sigmoid(
            jnp.dot(pooled, wl_ref[...], preferred_element_type=jnp.float32)
            + bl_ref[...])
        dmid = jnp.maximum(
            jnp.dot(pooled, wd1_ref[...], preferred_element_type=jnp.float32)
            + bd1_ref[...], 0.0)
        dom = (jnp.dot(dmid, wd2_ref[...], preferred_element_type=jnp.float32)
               + bd2_ref[...])
        col = lax.broadcasted_iota(jnp.int32, (G, 128), 1)
        o_ref[...] = jnp.where(col == 0, lab, dom)


def _finale(s0, s1, ht, dinv2, b2r, batchf, cntr,
            Wlp, blp, Wd1, bd1r, Wd2p, bd2p):
    return pl.pallas_call(
        _finale_body,
        out_shape=jax.ShapeDtypeStruct((G, 128), jnp.float32),
        grid=(NBLK,),
        in_specs=[pl.BlockSpec((BLK, 128), lambda i: (i, 0)),
                  pl.BlockSpec((BLK, 128), lambda i: (i, 0)),
                  pl.BlockSpec((BLK, 128), lambda i: (i, 0)),
                  pl.BlockSpec((BLK, 1), lambda i: (i, 0)),
                  pl.BlockSpec((1, 128), lambda i: (0, 0)),
                  pl.BlockSpec((BLK, 1), lambda i: (i, 0)),
                  pl.BlockSpec((G, 1), lambda i: (0, 0)),
                  pl.BlockSpec((128, 128), lambda i: (0, 0)),
                  pl.BlockSpec((1, 128), lambda i: (0, 0)),
                  pl.BlockSpec((128, 64), lambda i: (0, 0)),
                  pl.BlockSpec((1, 64), lambda i: (0, 0)),
                  pl.BlockSpec((64, 128), lambda i: (0, 0)),
                  pl.BlockSpec((1, 128), lambda i: (0, 0))],
        out_specs=pl.BlockSpec((G, 128), lambda i: (0, 0)),
    )(s0, s1, ht, dinv2, b2r, batchf, cntr, Wlp, blp, Wd1, bd1r, Wd2p, bd2p)


def kernel(x, edge_index, batch, W1, b1, W2, b2, Wl, bl, Wd1, bd1, Wd2, bd2):
    npad = NP - N
    epad = EPAD - E
    src = edge_index[0]
    dst = edge_index[1]
    ar = jnp.arange(epad, dtype=jnp.int32)
    src_p = jnp.concatenate([src, N + ar % npad])
    dst_p = jnp.concatenate([dst, NP + ar % 16])
    batch_p = jnp.concatenate([batch, jnp.full((npad,), G, jnp.int32)])
    x_p = jnp.pad(x, ((0, npad), (0, 0)))

    deg, cnt = _stats(dst_p, batch_p)
    dinv2 = _rsqrt_tc(deg.reshape(NP, 1))

    h1t = _mm_scale(x_p, W1, dinv2)
    s1 = _edge_scatter(h1t, src_p, dst_p)
    h2t = _layer_mid(s1[:NP], s1[NP:], h1t, dinv2, b1[None, :], W2)
    s2 = _edge_scatter(h2t, src_p, dst_p)

    heads = _finale(
        s2[:NP], s2[NP:], h2t, dinv2, b2[None, :],
        batch_p.astype(jnp.float32).reshape(NP, 1), cnt.reshape(G, 1),
        jnp.pad(Wl, ((0, 0), (0, 127))), jnp.pad(bl[None, :], ((0, 0), (0, 127))),
        Wd1, bd1[None, :],
        jnp.pad(Wd2, ((0, 0), (1, 125))), jnp.pad(bd2[None, :], ((0, 0), (1, 125))),
    )
    return heads[:, 0:1], heads[:, 1:3]


# speedup vs baseline: 21.3895x; 1.5984x over previous
import jax
import jax.numpy as jnp
from jax import lax
from jax.experimental import pallas as pl
from jax.experimental.pallas import tpu as pltpu
from jax.experimental.pallas import tpu_sc as plsc

N = 10000
G = 512
E = 320000
NP = 10240
EPT = E // 32
CH = 128
NFULL = EPT // CH
TAIL = EPT - NFULL * CH
NC, NS = 2, 16
RPT = NP // NS
BLK = 256
NBLK = NP // BLK


def _sc_mesh():
    return plsc.VectorSubcoreMesh(core_axis_name="c", subcore_axis_name="s")


def _deg(dst):
    k = pl.kernel(
        _deg_full_body,
        out_type=jax.ShapeDtypeStruct((NC * NP,), jnp.float32),
        mesh=_sc_mesh(),
        scratch_types=[
            pltpu.VMEM_SHARED((NP,), jnp.float32),
            pltpu.VMEM((CH,), jnp.int32),
            pltpu.VMEM((TAIL,), jnp.int32),
            pltpu.VMEM((CH,), jnp.float32),
            pltpu.VMEM((CH,), jnp.float32),
            pltpu.VMEM((RPT,), jnp.float32),
        ],
    )
    return k(dst)


def _deg_full_body(dst_hbm, deg_hbm, deg_sh, idx_i, tail_i, ones_v, half_v,
                   work_v):
    c = lax.axis_index("c")
    s = lax.axis_index("s")

    def fill(i, _):
        ones_v[pl.ds(i * 16, 16)] = jnp.full((16,), 1.0, jnp.float32)
        half_v[pl.ds(i * 16, 16)] = jnp.full((16,), 0.5, jnp.float32)
        return 0
    lax.fori_loop(0, CH // 16, fill, 0)

    base = s * RPT
    for k in range(RPT // CH):
        pltpu.sync_copy(half_v, deg_sh.at[pl.ds(base + k * CH, CH)])
    plsc.subcore_barrier()

    estart = (c * NS + s) * EPT

    def body(j, _):
        b = estart + j * CH
        pltpu.sync_copy(dst_hbm.at[pl.ds(b, CH)], idx_i)
        pltpu.sync_copy(ones_v, deg_sh.at[idx_i], add=True)
        return 0
    lax.fori_loop(0, NFULL, body, 0)
    pltpu.sync_copy(dst_hbm.at[pl.ds(estart + NFULL * CH, TAIL)], tail_i)
    pltpu.sync_copy(ones_v.at[pl.ds(0, TAIL)], deg_sh.at[tail_i], add=True)
    plsc.subcore_barrier()

    pltpu.sync_copy(deg_sh.at[pl.ds(base, RPT)], work_v)
    pltpu.sync_copy(work_v, deg_hbm.at[pl.ds(c * NP + base, RPT)])


def _scatter_body(h_hbm, src_hbm, dst_hbm, out_hbm, acc_sh,
                  src0, src1, dst0, dst1, rows0, rows1, tsrc, tdst, trows,
                  gs0, gs1, ss0, ss1, tsem):
    c = lax.axis_index("c")
    s = lax.axis_index("s")

    def zr(r, _):
        for k in range(8):
            rows0[r, pl.ds(k * 16, 16)] = jnp.zeros((16,), jnp.float32)
        return 0
    lax.fori_loop(0, CH, zr, 0)

    base = s * RPT
    for k in range(RPT // CH):
        pltpu.sync_copy(rows0, acc_sh.at[pl.ds(base + k * CH, CH)])
    plsc.subcore_barrier()

    estart = (c * NS + s) * EPT

    pltpu.sync_copy(src_hbm.at[pl.ds(estart, CH)], src0)
    pltpu.sync_copy(dst_hbm.at[pl.ds(estart, CH)], dst0)
    g0 = pltpu.async_copy(h_hbm.at[src0], rows0, gs0)

    def body(t, _):
        j = 2 * t

        @pl.when(t > 0)
        def _():
            pltpu.make_async_copy(rows1, acc_sh.at[dst1], ss1).wait()

        b1 = estart + (j + 1) * CH
        pltpu.sync_copy(src_hbm.at[pl.ds(b1, CH)], src1)
        pltpu.sync_copy(dst_hbm.at[pl.ds(b1, CH)], dst1)
        pltpu.async_copy(h_hbm.at[src1], rows1, gs1)

        pltpu.make_async_copy(h_hbm.at[src0], rows0, gs0).wait()
        pltpu.async_copy(rows0, acc_sh.at[dst0], ss0, add=True)

        @pl.when(t < NFULL // 2 - 1)
        def _():
            pltpu.make_async_copy(rows0, acc_sh.at[dst0], ss0).wait()
            b2 = estart + (j + 2) * CH
            pltpu.sync_copy(src_hbm.at[pl.ds(b2, CH)], src0)
            pltpu.sync_copy(dst_hbm.at[pl.ds(b2, CH)], dst0)
            pltpu.async_copy(h_hbm.at[src0], rows0, gs0)

        pltpu.make_async_copy(h_hbm.at[src1], rows1, gs1).wait()
        pltpu.async_copy(rows1, acc_sh.at[dst1], ss1, add=True)
        return 0
    lax.fori_loop(0, NFULL // 2, body, 0)
    pltpu.make_async_copy(rows0, acc_sh.at[dst0], ss0).wait()

    tb = estart + NFULL * CH
    pltpu.sync_copy(src_hbm.at[pl.ds(tb, TAIL)], tsrc)
    pltpu.sync_copy(dst_hbm.at[pl.ds(tb, TAIL)], tdst)
    pltpu.async_copy(h_hbm.at[tsrc], trows, tsem).wait()
    pltpu.sync_copy(trows, acc_sh.at[tdst], add=True)

    pltpu.make_async_copy(rows1, acc_sh.at[dst1], ss1).wait()
    plsc.subcore_barrier()

    ob = c * NP + base
    for k in range(RPT // CH):
        pltpu.sync_copy(acc_sh.at[pl.ds(base + k * CH, CH)], rows0)
        pltpu.sync_copy(rows0, out_hbm.at[pl.ds(ob + k * CH, CH)])


def _edge_scatter(h, src, dst):
    k = pl.kernel(
        _scatter_body,
        out_type=jax.ShapeDtypeStruct((NC * NP, 128), jnp.float32),
        mesh=_sc_mesh(),
        scratch_types=[
            pltpu.VMEM_SHARED((NP, 128), jnp.float32),
            pltpu.VMEM((CH,), jnp.int32),
            pltpu.VMEM((CH,), jnp.int32),
            pltpu.VMEM((CH,), jnp.int32),
            pltpu.VMEM((CH,), jnp.int32),
            pltpu.VMEM((CH, 128), jnp.float32),
            pltpu.VMEM((CH, 128), jnp.float32),
            pltpu.VMEM((TAIL,), jnp.int32),
            pltpu.VMEM((TAIL,), jnp.int32),
            pltpu.VMEM((TAIL, 128), jnp.float32),
            pltpu.SemaphoreType.DMA,
            pltpu.SemaphoreType.DMA,
            pltpu.SemaphoreType.DMA,
            pltpu.SemaphoreType.DMA,
            pltpu.SemaphoreType.DMA,
        ],
    )
    return k(h, src, dst)


def _mm_scale_body(x_ref, w_ref, d0_ref, d1_ref, o_ref, dv_ref):
    dinv = lax.rsqrt(d0_ref[...] + d1_ref[...])
    dv_ref[...] = dinv
    o_ref[...] = jnp.dot(x_ref[...], w_ref[...],
                         preferred_element_type=jnp.float32) * dinv


def _mm_scale(x_p, W, deg2):
    return pl.pallas_call(
        _mm_scale_body,
        out_shape=(jax.ShapeDtypeStruct((NP, 128), jnp.float32),
                   jax.ShapeDtypeStruct((NP, 1), jnp.float32)),
        grid=(NBLK,),
        in_specs=[pl.BlockSpec((BLK, 128), lambda i: (i, 0)),
                  pl.BlockSpec((128, 128), lambda i: (0, 0)),
                  pl.BlockSpec((BLK, 1), lambda i: (i, 0)),
                  pl.BlockSpec((BLK, 1), lambda i: (i + NBLK, 0))],
        out_specs=(pl.BlockSpec((BLK, 128), lambda i: (i, 0)),
                   pl.BlockSpec((BLK, 1), lambda i: (i, 0))),
    )(x_p, W, deg2, deg2)


def _layer_mid_body(s0_ref, s1_ref, h_ref, d_ref, b_ref, w_ref, o_ref):
    i = pl.program_id(0)
    tot = s0_ref[...] + s1_ref[...] + h_ref[...]
    act = jnp.maximum(d_ref[...] * tot + b_ref[...], 0.0)
    row = i * BLK + lax.broadcasted_iota(jnp.int32, (BLK, 128), 0)
    act = jnp.where(row < N, act, 0.0)
    o_ref[...] = jnp.dot(act, w_ref[...],
                         preferred_element_type=jnp.float32) * d_ref[...]


def _layer_mid(s, ht, dinv2, b1r, W2):
    return pl.pallas_call(
        _layer_mid_body,
        out_shape=jax.ShapeDtypeStruct((NP, 128), jnp.float32),
        grid=(NBLK,),
        in_specs=[pl.BlockSpec((BLK, 128), lambda i: (i, 0)),
                  pl.BlockSpec((BLK, 128), lambda i: (i + NBLK, 0)),
                  pl.BlockSpec((BLK, 128), lambda i: (i, 0)),
                  pl.BlockSpec((BLK, 1), lambda i: (i, 0)),
                  pl.BlockSpec((1, 128), lambda i: (0, 0)),
                  pl.BlockSpec((128, 128), lambda i: (0, 0))],
        out_specs=pl.BlockSpec((BLK, 128), lambda i: (i, 0)),
    )(s, s, ht, dinv2, b1r, W2)


def _finale_body(s0_ref, s1_ref, h_ref, d_ref, b_ref, bt_ref,
                 wl_ref, bl_ref, wd1_ref, bd1_ref, wd2_ref, bd2_ref,
                 o_ref, cacc):
    i = pl.program_id(0)
    tot = s0_ref[...] + s1_ref[...] + h_ref[...]
    act = jnp.maximum(d_ref[...] * tot + b_ref[...], 0.0)
    gid = lax.broadcasted_iota(jnp.int32, (BLK, G), 1)
    onehot = jnp.where(bt_ref[...].astype(jnp.int32) == gid, 1.0, 0.0)
    part = lax.dot_general(onehot, act, (((0,), (0,)), ((), ())),
                           preferred_element_type=jnp.float32)
    cpart = lax.dot_general(onehot, jnp.ones((BLK, 128), jnp.float32),
                            (((0,), (0,)), ((), ())),
                            preferred_element_type=jnp.float32)

    @pl.when(i == 0)
    def _():
        o_ref[...] = jnp.zeros_like(o_ref)
        cacc[...] = jnp.zeros_like(cacc)

    o_ref[...] += part
    cacc[...] += cpart

    @pl.when(i == NBLK - 1)
    def _():
        pooled = o_ref[...] / jnp.maximum(cacc[...], 1.0)
        lab = jax.nn.sigmoid(
            jnp.dot(pooled, wl_ref[...], preferred_element_type=jnp.float32)
            + bl_ref[...])
        dmid = jnp.maximum(
            jnp.dot(pooled, wd1_ref[...], preferred_element_type=jnp.float32)
            + bd1_ref[...], 0.0)
        dom = (jnp.dot(dmid, wd2_ref[...], preferred_element_type=jnp.float32)
               + bd2_ref[...])
        col = lax.broadcasted_iota(jnp.int32, (G, 128), 1)
        o_ref[...] = jnp.where(col == 0, lab, dom)


def _finale(s, ht, dinv2, b2r, batchf, Wlp, blp, Wd1, bd1r, Wd2p, bd2p):
    return pl.pallas_call(
        _finale_body,
        out_shape=jax.ShapeDtypeStruct((G, 128), jnp.float32),
        grid=(NBLK,),
        in_specs=[pl.BlockSpec((BLK, 128), lambda i: (i, 0)),
                  pl.BlockSpec((BLK, 128), lambda i: (i + NBLK, 0)),
                  pl.BlockSpec((BLK, 128), lambda i: (i, 0)),
                  pl.BlockSpec((BLK, 1), lambda i: (i, 0)),
                  pl.BlockSpec((1, 128), lambda i: (0, 0)),
                  pl.BlockSpec((BLK, 1), lambda i: (i, 0)),
                  pl.BlockSpec((128, 128), lambda i: (0, 0)),
                  pl.BlockSpec((1, 128), lambda i: (0, 0)),
                  pl.BlockSpec((128, 64), lambda i: (0, 0)),
                  pl.BlockSpec((1, 64), lambda i: (0, 0)),
                  pl.BlockSpec((64, 128), lambda i: (0, 0)),
                  pl.BlockSpec((1, 128), lambda i: (0, 0))],
        out_specs=pl.BlockSpec((G, 128), lambda i: (0, 0)),
        scratch_shapes=[pltpu.VMEM((G, 128), jnp.float32)],
    )(s, s, ht, dinv2, b2r, batchf, Wlp, blp, Wd1, bd1r, Wd2p, bd2p)


def kernel(x, edge_index, batch, W1, b1, W2, b2, Wl, bl, Wd1, bd1, Wd2, bd2):
    npad = NP - N
    src = edge_index[0]
    dst = edge_index[1]
    batch_p = jnp.concatenate([batch, jnp.full((npad,), G, jnp.int32)])
    x_p = jnp.pad(x, ((0, npad), (0, 0)))

    deg = _deg(dst)
    h1t, dinv2 = _mm_scale(x_p, W1, deg.reshape(NC * NP, 1))
    s1 = _edge_scatter(h1t, src, dst)
    h2t = _layer_mid(s1, h1t, dinv2, b1[None, :], W2)
    s2 = _edge_scatter(h2t, src, dst)

    heads = _finale(
        s2, h2t, dinv2, b2[None, :],
        batch_p.astype(jnp.float32).reshape(NP, 1),
        jnp.pad(Wl, ((0, 0), (0, 127))), jnp.pad(bl[None, :], ((0, 0), (0, 127))),
        Wd1, bd1[None, :],
        jnp.pad(Wd2, ((0, 0), (1, 125))), jnp.pad(bd2[None, :], ((0, 0), (1, 125))),
    )
    return heads[:, 0:1], heads[:, 1:3]


# speedup vs baseline: 22.9409x; 1.0725x over previous
import jax
import jax.numpy as jnp
from jax import lax
from jax.experimental import pallas as pl
from jax.experimental.pallas import tpu as pltpu
from jax.experimental.pallas import tpu_sc as plsc

N = 10000
G = 512
E = 320000
NP = 10240
EPT = E // 32
CH = 128
NFULL = EPT // CH
TAIL = EPT - NFULL * CH
NC, NS = 2, 16
RPT = NP // NS
BLK = 256
NBLK = NP // BLK


def _sc_mesh():
    return plsc.VectorSubcoreMesh(core_axis_name="c", subcore_axis_name="s")


DB = 3


def _deg_full_body(dst_hbm, deg_hbm, deg_sh,
                   idx0, idx1, idx2, tail_i, ones_v, half_v, work_v,
                   ss0, ss1, ss2):
    c = lax.axis_index("c")
    s = lax.axis_index("s")
    idxs = (idx0, idx1, idx2)
    sss = (ss0, ss1, ss2)

    def fill(i, _):
        ones_v[pl.ds(i * 16, 16)] = jnp.full((16,), 1.0, jnp.float32)
        half_v[pl.ds(i * 16, 16)] = jnp.full((16,), 0.5, jnp.float32)
        return 0
    lax.fori_loop(0, CH // 16, fill, 0)

    base = s * RPT
    for k in range(RPT // CH):
        pltpu.sync_copy(half_v, deg_sh.at[pl.ds(base + k * CH, CH)])
    plsc.subcore_barrier()

    estart = (c * NS + s) * EPT

    def body(t, _):
        for b in range(DB):
            @pl.when(t > 0)
            def _(b=b):
                pltpu.make_async_copy(ones_v, deg_sh.at[idxs[b]], sss[b]).wait()
            pltpu.sync_copy(dst_hbm.at[pl.ds(estart + (DB * t + b) * CH, CH)],
                            idxs[b])
            pltpu.async_copy(ones_v, deg_sh.at[idxs[b]], sss[b], add=True)
        return 0
    lax.fori_loop(0, NFULL // DB, body, 0)
    for b in range(DB):
        pltpu.make_async_copy(ones_v, deg_sh.at[idxs[b]], sss[b]).wait()
    pltpu.sync_copy(dst_hbm.at[pl.ds(estart + NFULL * CH, TAIL)], tail_i)
    pltpu.sync_copy(ones_v.at[pl.ds(0, TAIL)], deg_sh.at[tail_i], add=True)
    plsc.subcore_barrier()

    pltpu.sync_copy(deg_sh.at[pl.ds(base, RPT)], work_v)
    pltpu.sync_copy(work_v, deg_hbm.at[pl.ds(c * NP + base, RPT)])


def _deg(dst):
    k = pl.kernel(
        _deg_full_body,
        out_type=jax.ShapeDtypeStruct((NC * NP,), jnp.float32),
        mesh=_sc_mesh(),
        scratch_types=[
            pltpu.VMEM_SHARED((NP,), jnp.float32),
            pltpu.VMEM((CH,), jnp.int32),
            pltpu.VMEM((CH,), jnp.int32),
            pltpu.VMEM((CH,), jnp.int32),
            pltpu.VMEM((TAIL,), jnp.int32),
            pltpu.VMEM((CH,), jnp.float32),
            pltpu.VMEM((CH,), jnp.float32),
            pltpu.VMEM((RPT,), jnp.float32),
            pltpu.SemaphoreType.DMA,
            pltpu.SemaphoreType.DMA,
            pltpu.SemaphoreType.DMA,
        ],
    )
    return k(dst)


SB = 3
AR = 10112
ARPT = AR // NS


def _scatter_body(h_hbm, src_hbm, dst_hbm, out_hbm, acc_sh, *rest):
    srcs = rest[0:SB]
    dsts = rest[SB:2 * SB]
    rows = rest[2 * SB:3 * SB]
    tsrc, tdst = rest[3 * SB:3 * SB + 2]
    gss = rest[3 * SB + 2:4 * SB + 2]
    sss = rest[4 * SB + 2:5 * SB + 2]
    tsem = rest[5 * SB + 2]
    c = lax.axis_index("c")
    s = lax.axis_index("s")

    def zr(r, _):
        for k in range(8):
            rows[0][r, pl.ds(k * 16, 16)] = jnp.zeros((16,), jnp.float32)
        return 0
    lax.fori_loop(0, CH, zr, 0)

    base = s * ARPT
    for k in range(ARPT // CH):
        pltpu.sync_copy(rows[0], acc_sh.at[pl.ds(base + k * CH, CH)])
    pltpu.sync_copy(rows[0].at[pl.ds(0, ARPT % CH)],
                    acc_sh.at[pl.ds(base + (ARPT // CH) * CH, ARPT % CH)])
    plsc.subcore_barrier()

    estart = (c * NS + s) * EPT

    def body(t, _):
        for b in range(SB):
            @pl.when(t > 0)
            def _(b=b):
                pltpu.make_async_copy(rows[b], acc_sh.at[dsts[b]], sss[b]).wait()
            bo = estart + (SB * t + b) * CH
            pltpu.sync_copy(src_hbm.at[pl.ds(bo, CH)], srcs[b])
            pltpu.sync_copy(dst_hbm.at[pl.ds(bo, CH)], dsts[b])
            pltpu.async_copy(h_hbm.at[srcs[b]], rows[b], gss[b])
        for b in range(SB):
            pltpu.make_async_copy(h_hbm.at[srcs[b]], rows[b], gss[b]).wait()
            pltpu.async_copy(rows[b], acc_sh.at[dsts[b]], sss[b], add=True)
        return 0
    lax.fori_loop(0, NFULL // SB, body, 0)
    for b in range(SB):
        pltpu.make_async_copy(rows[b], acc_sh.at[dsts[b]], sss[b]).wait()

    tb = estart + NFULL * CH
    pltpu.sync_copy(src_hbm.at[pl.ds(tb, TAIL)], tsrc)
    pltpu.sync_copy(dst_hbm.at[pl.ds(tb, TAIL)], tdst)
    pltpu.async_copy(h_hbm.at[tsrc], rows[0].at[pl.ds(0, TAIL)], tsem).wait()
    pltpu.sync_copy(rows[0].at[pl.ds(0, TAIL)], acc_sh.at[tdst], add=True)
    plsc.subcore_barrier()

    ob = c * NP + base
    for k in range(ARPT // CH):
        pltpu.sync_copy(acc_sh.at[pl.ds(base + k * CH, CH)], rows[0])
        pltpu.sync_copy(rows[0], out_hbm.at[pl.ds(ob + k * CH, CH)])
    pltpu.sync_copy(acc_sh.at[pl.ds(base + (ARPT // CH) * CH, ARPT % CH)],
                    rows[0].at[pl.ds(0, ARPT % CH)])
    pltpu.sync_copy(rows[0].at[pl.ds(0, ARPT % CH)],
                    out_hbm.at[pl.ds(ob + (ARPT // CH) * CH, ARPT % CH)])


def _edge_scatter(h, src, dst):
    k = pl.kernel(
        _scatter_body,
        out_type=jax.ShapeDtypeStruct((NC * NP, 128), jnp.float32),
        mesh=_sc_mesh(),
        scratch_types=(
            [pltpu.VMEM_SHARED((AR, 128), jnp.float32)]
            + [pltpu.VMEM((CH,), jnp.int32) for _ in range(2 * SB)]
            + [pltpu.VMEM((CH, 128), jnp.float32) for _ in range(SB)]
            + [pltpu.VMEM((TAIL,), jnp.int32) for _ in range(2)]
            + [pltpu.SemaphoreType.DMA for _ in range(2 * SB + 1)]
        ),
    )
    return k(h, src, dst)


def _mm_scale_body(x_ref, w_ref, d0_ref, d1_ref, o_ref, dv_ref):
    dinv = lax.rsqrt(d0_ref[...] + d1_ref[...])
    dv_ref[...] = dinv
    o_ref[...] = jnp.dot(x_ref[...], w_ref[...],
                         preferred_element_type=jnp.float32) * dinv


def _mm_scale(x_p, W, deg2):
    return pl.pallas_call(
        _mm_scale_body,
        out_shape=(jax.ShapeDtypeStruct((NP, 128), jnp.float32),
                   jax.ShapeDtypeStruct((NP, 1), jnp.float32)),
        grid=(NBLK,),
        in_specs=[pl.BlockSpec((BLK, 128), lambda i: (i, 0)),
                  pl.BlockSpec((128, 128), lambda i: (0, 0)),
                  pl.BlockSpec((BLK, 1), lambda i: (i, 0)),
                  pl.BlockSpec((BLK, 1), lambda i: (i + NBLK, 0))],
        out_specs=(pl.BlockSpec((BLK, 128), lambda i: (i, 0)),
                   pl.BlockSpec((BLK, 1), lambda i: (i, 0))),
    )(x_p, W, deg2, deg2)


def _layer_mid_body(s0_ref, s1_ref, h_ref, d_ref, b_ref, w_ref, o_ref):
    i = pl.program_id(0)
    tot = s0_ref[...] + s1_ref[...] + h_ref[...]
    act = jnp.maximum(d_ref[...] * tot + b_ref[...], 0.0)
    row = i * BLK + lax.broadcasted_iota(jnp.int32, (BLK, 128), 0)
    act = jnp.where(row < N, act, 0.0)
    o_ref[...] = jnp.dot(act, w_ref[...],
                         preferred_element_type=jnp.float32) * d_ref[...]


def _layer_mid(s, ht, dinv2, b1r, W2):
    return pl.pallas_call(
        _layer_mid_body,
        out_shape=jax.ShapeDtypeStruct((NP, 128), jnp.float32),
        grid=(NBLK,),
        in_specs=[pl.BlockSpec((BLK, 128), lambda i: (i, 0)),
                  pl.BlockSpec((BLK, 128), lambda i: (i + NBLK, 0)),
                  pl.BlockSpec((BLK, 128), lambda i: (i, 0)),
                  pl.BlockSpec((BLK, 1), lambda i: (i, 0)),
                  pl.BlockSpec((1, 128), lambda i: (0, 0)),
                  pl.BlockSpec((128, 128), lambda i: (0, 0))],
        out_specs=pl.BlockSpec((BLK, 128), lambda i: (i, 0)),
    )(s, s, ht, dinv2, b1r, W2)


def _finale_body(s0_ref, s1_ref, h_ref, d_ref, b_ref, bt_ref,
                 wl_ref, bl_ref, wd1_ref, bd1_ref, wd2_ref, bd2_ref,
                 o_ref, cacc):
    i = pl.program_id(0)
    tot = s0_ref[...] + s1_ref[...] + h_ref[...]
    act = jnp.maximum(d_ref[...] * tot + b_ref[...], 0.0)
    row = i * BLK + lax.broadcasted_iota(jnp.int32, (BLK, 128), 0)
    act = jnp.where(row < N, act, 0.0)
    gid = lax.broadcasted_iota(jnp.int32, (BLK, G), 1)
    onehot = jnp.where(bt_ref[...].astype(jnp.int32) == gid, 1.0, 0.0)
    part = lax.dot_general(onehot, act, (((0,), (0,)), ((), ())),
                           preferred_element_type=jnp.float32)
    cpart = lax.dot_general(onehot, jnp.ones((BLK, 128), jnp.float32),
                            (((0,), (0,)), ((), ())),
                            preferred_element_type=jnp.float32)

    @pl.when(i == 0)
    def _():
        o_ref[...] = jnp.zeros_like(o_ref)
        cacc[...] = jnp.zeros_like(cacc)

    o_ref[...] += part
    cacc[...] += cpart

    @pl.when(i == NBLK - 1)
    def _():
        pooled = o_ref[...] / jnp.maximum(cacc[...], 1.0)
        lab = jax.nn.sigmoid(
            jnp.dot(pooled, wl_ref[...], preferred_element_type=jnp.float32)
            + bl_ref[...])
        dmid = jnp.maximum(
            jnp.dot(pooled, wd1_ref[...], preferred_element_type=jnp.float32)
            + bd1_ref[...], 0.0)
        dom = (jnp.dot(dmid, wd2_ref[...], preferred_element_type=jnp.float32)
               + bd2_ref[...])
        col = lax.broadcasted_iota(jnp.int32, (G, 128), 1)
        o_ref[...] = jnp.where(col == 0, lab, dom)


def _finale(s, ht, dinv2, b2r, batchf, Wlp, blp, Wd1, bd1r, Wd2p, bd2p):
    return pl.pallas_call(
        _finale_body,
        out_shape=jax.ShapeDtypeStruct((G, 128), jnp.float32),
        grid=(NBLK,),
        in_specs=[pl.BlockSpec((BLK, 128), lambda i: (i, 0)),
                  pl.BlockSpec((BLK, 128), lambda i: (i + NBLK, 0)),
                  pl.BlockSpec((BLK, 128), lambda i: (i, 0)),
                  pl.BlockSpec((BLK, 1), lambda i: (i, 0)),
                  pl.BlockSpec((1, 128), lambda i: (0, 0)),
                  pl.BlockSpec((BLK, 1), lambda i: (i, 0)),
                  pl.BlockSpec((128, 128), lambda i: (0, 0)),
                  pl.BlockSpec((1, 128), lambda i: (0, 0)),
                  pl.BlockSpec((128, 64), lambda i: (0, 0)),
                  pl.BlockSpec((1, 64), lambda i: (0, 0)),
                  pl.BlockSpec((64, 128), lambda i: (0, 0)),
                  pl.BlockSpec((1, 128), lambda i: (0, 0))],
        out_specs=pl.BlockSpec((G, 128), lambda i: (0, 0)),
        scratch_shapes=[pltpu.VMEM((G, 128), jnp.float32)],
    )(s, s, ht, dinv2, b2r, batchf, Wlp, blp, Wd1, bd1r, Wd2p, bd2p)


def kernel(x, edge_index, batch, W1, b1, W2, b2, Wl, bl, Wd1, bd1, Wd2, bd2):
    npad = NP - N
    src = edge_index[0]
    dst = edge_index[1]
    batch_p = jnp.concatenate([batch, jnp.full((npad,), G, jnp.int32)])
    x_p = jnp.pad(x, ((0, npad), (0, 0)))

    deg = _deg(dst)
    h1t, dinv2 = _mm_scale(x_p, W1, deg.reshape(NC * NP, 1))
    s1 = _edge_scatter(h1t, src, dst)
    h2t = _layer_mid(s1, h1t, dinv2, b1[None, :], W2)
    s2 = _edge_scatter(h2t, src, dst)

    heads = _finale(
        s2, h2t, dinv2, b2[None, :],
        batch_p.astype(jnp.float32).reshape(NP, 1),
        jnp.pad(Wl, ((0, 0), (0, 127))), jnp.pad(bl[None, :], ((0, 0), (0, 127))),
        Wd1, bd1[None, :],
        jnp.pad(Wd2, ((0, 0), (1, 125))), jnp.pad(bd2[None, :], ((0, 0), (1, 125))),
    )
    return heads[:, 0:1], heads[:, 1:3]
